# baseline (speedup 1.0000x reference)
import functools

import jax
import jax.numpy as jnp
from jax import lax
from jax.experimental import pallas as pl
from jax.experimental.pallas import tpu as pltpu
from jax.experimental.pallas import tpu_sc as plsc

N = 10000
E = 320000
D = 128
H = 128
NT = 16
G = 64
NDIFF = 1000

NC = 2
NS = 16
NW = NC * NS
EPW = E // NW
CH = 80
NCHUNK = EPW // CH
ZB = 624
K = 3
SBLK = 15
NBLK = 8
NP = 10240
CPS = NP // NS

@functools.lru_cache(maxsize=None)
def _build_sc_kernels():
    mesh = plsc.VectorSubcoreMesh(core_axis_name="c", subcore_axis_name="s")
    prep = functools.partial(
        pl.kernel,
        out_type=(
            jax.ShapeDtypeStruct((NC * NP,), jnp.float32),
            jax.ShapeDtypeStruct((NP,), jnp.float32),
        ),
        mesh=mesh,
        compiler_params=pltpu.CompilerParams(needs_layout_passes=False),
        scratch_types=[
            pltpu.VMEM((EPW,), jnp.int32),
            pltpu.VMEM((NP,), jnp.float32),
            pltpu.VMEM((CPS,), jnp.float32),
            pltpu.VMEM((CPS,), jnp.float32),
            pltpu.VMEM((NDIFF,), jnp.int32),
            pltpu.VMEM((NP,), jnp.float32),
            pltpu.VMEM_SHARED((NS * NP,), jnp.float32),
        ],
    )(_prep_sc_body)
    msg = functools.partial(
        pl.kernel,
        out_type=jax.ShapeDtypeStruct((NC * N, D), jnp.float32),
        mesh=mesh,
        compiler_params=pltpu.CompilerParams(needs_layout_passes=False),
        scratch_types=[
            pltpu.VMEM((SBLK * CH,), jnp.int32),
            pltpu.VMEM((NCHUNK, CH), jnp.int32),
            pltpu.VMEM((SBLK * CH,), jnp.int32),
            pltpu.VMEM((K, CH, D), jnp.float32),
            pltpu.VMEM((NT,), jnp.float32),
            pltpu.VMEM((8, D), jnp.float32),
            pltpu.VMEM_SHARED((N, D), jnp.float32),
            pltpu.SemaphoreType.DMA,
            pltpu.SemaphoreType.DMA,
        ],
    )(_msg_sc_body)
    return prep, msg


def _prep_sc_body(dst_hbm, diff_hbm, deg_hbm, ind_hbm,
                  dstl, degl, tmp, acc, diffv, indv, sdeg):
    cid = lax.axis_index("c")
    sid = lax.axis_index("s")
    wid = cid * NS + sid
    ones = jnp.ones((16,), jnp.float32)
    zeros = jnp.zeros((16,), jnp.float32)

    def _z(i, carry):
        degl[pl.ds(16 * i, 16)] = zeros
        return carry
    lax.fori_loop(0, NP // 16, _z, 0)

    pltpu.sync_copy(dst_hbm.at[pl.ds(wid * EPW, EPW)], dstl)

    def _hist(j, carry):
        iv = dstl[pl.ds(16 * j, 16)]
        plsc.addupdate_scatter(degl, [iv], ones)
        return carry
    lax.fori_loop(0, EPW // 16, _hist, 0)

    pltpu.sync_copy(degl, sdeg.at[pl.ds(sid * NP, NP)])
    plsc.subcore_barrier()

    col = sid * CPS

    def _za(i, carry):
        acc[pl.ds(16 * i, 16)] = zeros
        return carry
    lax.fori_loop(0, CPS // 16, _za, 0)

    for j in range(NS):
        pltpu.sync_copy(sdeg.at[pl.ds(j * NP + col, CPS)], tmp)

        def _add(i, carry):
            acc[pl.ds(16 * i, 16)] = acc[pl.ds(16 * i, 16)] + tmp[pl.ds(16 * i, 16)]
            return carry
        lax.fori_loop(0, CPS // 16, _add, 0)

    pltpu.sync_copy(acc, deg_hbm.at[pl.ds(cid * NP + col, CPS)])

    @pl.when(jnp.logical_and(cid == 0, sid == 0))
    def _ind():
        def _zi(i, carry):
            indv[pl.ds(16 * i, 16)] = zeros
            return carry
        lax.fori_loop(0, NP // 16, _zi, 0)
        pltpu.sync_copy(diff_hbm, diffv)

        def _sc(j, carry):
            iv = diffv[pl.ds(16 * j, 16)]
            plsc.store_scatter(indv, [iv], ones)
            return carry
        lax.fori_loop(0, NDIFF // 16, _sc, 0)
        iv = diffv[pl.ds(NDIFF - 16, 16)]
        plsc.store_scatter(indv, [iv], ones)
        pltpu.sync_copy(indv, ind_hbm)


def _msg_sc_body(hn_hbm, src_hbm, dst_hbm, et_hbm, wtab_hbm, out_hbm,
                 srcb, dstv, etb, rows, wtabv, zbuf, macc, semg, sems):
    cid = lax.axis_index("c")
    sid = lax.axis_index("s")
    wid = cid * NS + sid

    pltpu.sync_copy(dst_hbm.at[wid], dstv)
    pltpu.sync_copy(wtab_hbm, wtabv)

    for r in range(8):
        for k in range(D // 16):
            zbuf[r, pl.ds(16 * k, 16)] = jnp.zeros((16,), jnp.float32)

    base = sid * ZB

    def _zero(i, carry):
        pltpu.sync_copy(zbuf, macc.at[pl.ds(base + i * 8, 8)])
        return carry
    lax.fori_loop(0, ZB // 8, _zero, 0)

    @pl.when(sid == NS - 1)
    def _zero_tail():
        pltpu.sync_copy(zbuf, macc.at[pl.ds(NS * ZB, 8)])
        pltpu.sync_copy(zbuf, macc.at[pl.ds(NS * ZB + 8, 8)])
    plsc.subcore_barrier()

    def _group(local_base, cbase, nk):
        def _fire(b, carry):
            pltpu.async_copy(
                hn_hbm.at[srcb.at[pl.ds(local_base + b * CH, CH)]],
                rows.at[b], semg)
            return carry
        lax.fori_loop(0, nk, _fire, 0)

        def _drain_g(b, carry):
            pltpu.make_async_copy(
                hn_hbm.at[srcb.at[pl.ds(local_base + b * CH, CH)]],
                rows.at[b], semg).wait()
            return carry
        lax.fori_loop(0, nk, _drain_g, 0)

        def _one(b, carry):
            def _scale(g, inner):
                etk = etb[pl.ds(local_base + b * CH + 16 * g, 16)]
                wk = plsc.load_gather(wtabv, [etk])
                for j in range(16):
                    s = wk[j]
                    r0 = 16 * g + j
                    for k in range(D // 16):
                        rows[b, r0, pl.ds(16 * k, 16)] = (
                            rows[b, r0, pl.ds(16 * k, 16)] * s)
                return inner
            lax.fori_loop(0, CH // 16, _scale, 0)
            pltpu.async_copy(rows.at[b], macc.at[dstv.at[cbase + b]], sems,
                             add=True)
            return carry
        lax.fori_loop(0, nk, _one, 0)

        def _drain_s(b, carry):
            pltpu.make_async_copy(rows.at[b], macc.at[dstv.at[cbase]],
                                  sems).wait()
            return carry
        lax.fori_loop(0, nk, _drain_s, 0)

    def _block(b7, carry):
        estart = wid * EPW + b7 * (SBLK * CH)
        pltpu.sync_copy(src_hbm.at[pl.ds(estart, SBLK * CH)], srcb)
        pltpu.sync_copy(et_hbm.at[pl.ds(estart, SBLK * CH)], etb)

        def _g5(g5, inner):
            _group(g5 * (K * CH), b7 * SBLK + g5 * K, K)
            return inner
        lax.fori_loop(0, SBLK // K, _g5, 0)
        return carry
    lax.fori_loop(0, NBLK, _block, 0)

    tstart = wid * EPW + NBLK * (SBLK * CH)
    pltpu.sync_copy(src_hbm.at[pl.ds(tstart, 5 * CH)], srcb.at[pl.ds(0, 5 * CH)])
    pltpu.sync_copy(et_hbm.at[pl.ds(tstart, 5 * CH)], etb.at[pl.ds(0, 5 * CH)])
    _group(0, NBLK * SBLK, K)
    _group(K * CH, NBLK * SBLK + K, 2)

    plsc.subcore_barrier()
    pltpu.sync_copy(macc.at[pl.ds(base, ZB)],
                    out_hbm.at[pl.ds(cid * N + base, ZB)])

    @pl.when(sid == NS - 1)
    def _copy_tail():
        pltpu.sync_copy(macc.at[pl.ds(NS * ZB, 16)],
                        out_hbm.at[pl.ds(cid * N + NS * ZB, 16)])


BR = 2000


def _pre_body(x_ref, w1t, b1, w2t, b2, hn_ref):
    h = jnp.dot(x_ref[...], w1t[...], preferred_element_type=jnp.float32) + b1[...]
    h = jnp.maximum(h, 0.0)
    h = jnp.dot(h, w2t[...], preferred_element_type=jnp.float32) + b2[...]
    mu = jnp.mean(h, axis=-1, keepdims=True)
    var = jnp.mean((h - mu) ** 2, axis=-1, keepdims=True)
    hn_ref[...] = (h - mu) * lax.rsqrt(var + 1e-5)


_pre = pl.pallas_call(
    _pre_body,
    grid=(N // BR,),
    in_specs=[
        pl.BlockSpec((BR, D), lambda i: (i, 0)),
        pl.BlockSpec((D, H), lambda i: (0, 0)),
        pl.BlockSpec((1, H), lambda i: (0, 0)),
        pl.BlockSpec((H, H), lambda i: (0, 0)),
        pl.BlockSpec((1, H), lambda i: (0, 0)),
    ],
    out_specs=pl.BlockSpec((BR, H), lambda i: (i, 0)),
    out_shape=jax.ShapeDtypeStruct((N, H), jnp.float32),
)


def _gru_body(m0, m1, d0, d1, hn_ref, wih, whh, bih, bhh, h_out, hn_out):
    deg = jnp.maximum(d0[...] + d1[...], 1.0)
    m = (m0[...] + m1[...]) / deg
    hn = hn_ref[...]
    gi = jnp.dot(m, wih[...], preferred_element_type=jnp.float32) + bih[...]
    gh = jnp.dot(hn, whh[...], preferred_element_type=jnp.float32) + bhh[...]
    r = jax.nn.sigmoid(gi[:, :H] + gh[:, :H])
    z = jax.nn.sigmoid(gi[:, H:2 * H] + gh[:, H:2 * H])
    n = jnp.tanh(gi[:, 2 * H:] + r * gh[:, 2 * H:])
    h = (1.0 - z) * n + z * hn
    h_out[...] = h
    mu = jnp.mean(h, axis=-1, keepdims=True)
    var = jnp.mean((h - mu) ** 2, axis=-1, keepdims=True)
    hn_out[...] = (h - mu) * lax.rsqrt(var + 1e-5)


_gru = pl.pallas_call(
    _gru_body,
    grid=(N // BR,),
    in_specs=[
        pl.BlockSpec((BR, D), lambda i: (i, 0)),
        pl.BlockSpec((BR, D), lambda i: (i, 0)),
        pl.BlockSpec((BR, 1), lambda i: (i, 0)),
        pl.BlockSpec((BR, 1), lambda i: (i, 0)),
        pl.BlockSpec((BR, D), lambda i: (i, 0)),
        pl.BlockSpec((H, 3 * H), lambda i: (0, 0)),
        pl.BlockSpec((H, 3 * H), lambda i: (0, 0)),
        pl.BlockSpec((1, 3 * H), lambda i: (0, 0)),
        pl.BlockSpec((1, 3 * H), lambda i: (0, 0)),
    ],
    out_specs=[
        pl.BlockSpec((BR, H), lambda i: (i, 0)),
        pl.BlockSpec((BR, H), lambda i: (i, 0)),
    ],
    out_shape=[
        jax.ShapeDtypeStruct((N, H), jnp.float32),
        jax.ShapeDtypeStruct((N, H), jnp.float32),
    ],
)


def _readout_body(h, ind, bt, wp, f1t, b1, f2t, b2, f3t, b3, out, accm, acca):
    i = pl.program_id(0)

    @pl.when(i == 0)
    def _init():
        accm[...] = jnp.zeros_like(accm)
        acca[...] = jnp.zeros_like(acca)

    alpha = 1.0 + wp[...] * ind[...]
    bio = lax.broadcasted_iota(jnp.int32, (BR, G), 1)
    oh = (bio == bt[...]).astype(jnp.float32)
    accm[...] += lax.dot_general(oh, alpha * h[...],
                                 (((0,), (0,)), ((), ())),
                                 preferred_element_type=jnp.float32)
    acca[...] += lax.dot_general(oh, alpha,
                                 (((0,), (0,)), ((), ())),
                                 preferred_element_type=jnp.float32)

    @pl.when(i == pl.num_programs(0) - 1)
    def _fin():
        xg = accm[...] / acca[...]
        o = jnp.dot(xg, f1t[...], preferred_element_type=jnp.float32) + b1[...]
        o = jnp.maximum(o, 0.0)
        o = jnp.dot(o, f2t[...], preferred_element_type=jnp.float32) + b2[...]
        o = jnp.maximum(o, 0.0)
        out[...] = jnp.dot(o, f3t[...], preferred_element_type=jnp.float32) + b3[...]


_readout = pl.pallas_call(
    _readout_body,
    grid=(N // BR,),
    in_specs=[
        pl.BlockSpec((BR, D), lambda i: (i, 0)),
        pl.BlockSpec((BR, 1), lambda i: (i, 0)),
        pl.BlockSpec((BR, 1), lambda i: (i, 0)),
        pl.BlockSpec((1, 1), lambda i: (0, 0)),
        pl.BlockSpec((H, H // 2), lambda i: (0, 0)),
        pl.BlockSpec((1, H // 2), lambda i: (0, 0)),
        pl.BlockSpec((H // 2, H // 4), lambda i: (0, 0)),
        pl.BlockSpec((1, H // 4), lambda i: (0, 0)),
        pl.BlockSpec((H // 4, 2), lambda i: (0, 0)),
        pl.BlockSpec((1, 2), lambda i: (0, 0)),
    ],
    out_specs=pl.BlockSpec((G, 2), lambda i: (0, 0)),
    out_shape=jax.ShapeDtypeStruct((G, 2), jnp.float32),
    scratch_shapes=[
        pltpu.VMEM((G, D), jnp.float32),
        pltpu.VMEM((G, 1), jnp.float32),
    ],
)


def kernel(x, edge_index, edge_type, diff_idx, batch,
           lin1_W, lin1_b, lin2_W, lin2_b, edge_type_weight,
           gru_Wih, gru_Whh, gru_bih, gru_bhh,
           fc1_W, fc1_b, fc2_W, fc2_b, fc3_W, fc3_b, w_imp):
    src1 = edge_index[0]
    dst3 = edge_index[1].reshape(NW, NCHUNK, CH)
    et1 = edge_type
    wtab = jnp.logaddexp(0.0, edge_type_weight).astype(jnp.float32)

    _prep_sc, _msg_sc = _build_sc_kernels()
    degp, ind = _prep_sc(edge_index[1], diff_idx)
    d0 = degp[:N].reshape(N, 1)
    d1 = degp[NP:NP + N].reshape(N, 1)
    indc = ind[:N].reshape(N, 1)

    hn = _pre(x, lin1_W.T, lin1_b.reshape(1, H), lin2_W.T, lin2_b.reshape(1, H))

    wih_t = gru_Wih.T
    whh_t = gru_Whh.T
    bih = gru_bih.reshape(1, 3 * H)
    bhh = gru_bhh.reshape(1, 3 * H)

    h = hn
    for _ in range(3):
        mflat = _msg_sc(hn, src1, dst3, et1, wtab)
        h, hn = _gru(mflat[:N], mflat[N:], d0, d1, hn, wih_t, whh_t, bih, bhh)

    wp = jnp.logaddexp(0.0, w_imp).reshape(1, 1)
    return _readout(h, indc, batch.reshape(N, 1), wp,
                    fc1_W.T, fc1_b.reshape(1, H // 2),
                    fc2_W.T, fc2_b.reshape(1, H // 4),
                    fc3_W.T, fc3_b.reshape(1, 2))


# speedup vs baseline: 2.5667x; 2.5667x over previous
import functools

import jax
import jax.numpy as jnp
from jax import lax
from jax.experimental import pallas as pl
from jax.experimental.pallas import tpu as pltpu
from jax.experimental.pallas import tpu_sc as plsc

N = 10000
E = 320000
D = 128
H = 128
NT = 16
G = 64
NDIFF = 1000

NC = 2
NS = 16
NW = NC * NS
EPW = E // NW
CH = 80
NCHUNK = EPW // CH
ZB = 624
K = 3
SBLK = 15
NBLK = 8
NP = 10240
CPS = NP // NS

@functools.lru_cache(maxsize=None)
def _build_sc_kernels():
    mesh = plsc.VectorSubcoreMesh(core_axis_name="c", subcore_axis_name="s")
    prep = functools.partial(
        pl.kernel,
        out_type=(
            jax.ShapeDtypeStruct((NC * NP,), jnp.float32),
            jax.ShapeDtypeStruct((NP,), jnp.float32),
        ),
        mesh=mesh,
        compiler_params=pltpu.CompilerParams(needs_layout_passes=False),
        scratch_types=[
            pltpu.VMEM((EPW,), jnp.int32),
            pltpu.VMEM((NP,), jnp.float32),
            pltpu.VMEM((CPS,), jnp.float32),
            pltpu.VMEM((CPS,), jnp.float32),
            pltpu.VMEM((NDIFF,), jnp.int32),
            pltpu.VMEM((NP,), jnp.float32),
            pltpu.VMEM_SHARED((NS * NP,), jnp.float32),
        ],
    )(_prep_sc_body)
    msg = functools.partial(
        pl.kernel,
        out_type=jax.ShapeDtypeStruct((NC * N, D), jnp.float32),
        mesh=mesh,
        compiler_params=pltpu.CompilerParams(needs_layout_passes=False),
        scratch_types=[
            pltpu.VMEM((SBLK * CH,), jnp.int32),
            pltpu.VMEM((NCHUNK, CH), jnp.int32),
            pltpu.VMEM((SBLK * CH,), jnp.int32),
            pltpu.VMEM((K, CH, D), jnp.float32),
            pltpu.VMEM((NT,), jnp.float32),
            pltpu.VMEM((8, D), jnp.float32),
            pltpu.VMEM_SHARED((N, D), jnp.float32),
            pltpu.SemaphoreType.DMA,
            pltpu.SemaphoreType.DMA,
        ],
    )(_msg_sc_body)
    return prep, msg


def _prep_sc_body(dst_hbm, diff_hbm, deg_hbm, ind_hbm,
                  dstl, degl, tmp, acc, diffv, indv, sdeg):
    cid = lax.axis_index("c")
    sid = lax.axis_index("s")
    wid = cid * NS + sid
    ones = jnp.ones((16,), jnp.float32)
    zeros = jnp.zeros((16,), jnp.float32)

    def _z(i, carry):
        degl[pl.ds(16 * i, 16)] = zeros
        return carry
    lax.fori_loop(0, NP // 16, _z, 0)

    pltpu.sync_copy(dst_hbm.at[pl.ds(wid * EPW, EPW)], dstl)

    def _hist(j, carry):
        iv = dstl[pl.ds(16 * j, 16)]
        plsc.addupdate_scatter(degl, [iv], ones)
        return carry
    lax.fori_loop(0, EPW // 16, _hist, 0)

    pltpu.sync_copy(degl, sdeg.at[pl.ds(sid * NP, NP)])
    plsc.subcore_barrier()

    col = sid * CPS

    def _za(i, carry):
        acc[pl.ds(16 * i, 16)] = zeros
        return carry
    lax.fori_loop(0, CPS // 16, _za, 0)

    for j in range(NS):
        pltpu.sync_copy(sdeg.at[pl.ds(j * NP + col, CPS)], tmp)

        def _add(i, carry):
            acc[pl.ds(16 * i, 16)] = acc[pl.ds(16 * i, 16)] + tmp[pl.ds(16 * i, 16)]
            return carry
        lax.fori_loop(0, CPS // 16, _add, 0)

    pltpu.sync_copy(acc, deg_hbm.at[pl.ds(cid * NP + col, CPS)])

    @pl.when(jnp.logical_and(cid == 0, sid == 0))
    def _ind():
        def _zi(i, carry):
            indv[pl.ds(16 * i, 16)] = zeros
            return carry
        lax.fori_loop(0, NP // 16, _zi, 0)
        pltpu.sync_copy(diff_hbm, diffv)

        def _sc(j, carry):
            iv = diffv[pl.ds(16 * j, 16)]
            plsc.store_scatter(indv, [iv], ones)
            return carry
        lax.fori_loop(0, NDIFF // 16, _sc, 0)
        iv = diffv[pl.ds(NDIFF - 16, 16)]
        plsc.store_scatter(indv, [iv], ones)
        pltpu.sync_copy(indv, ind_hbm)


def _msg_sc_body(hn_hbm, src_hbm, dst_hbm, et_hbm, wtab_hbm, out_hbm,
                 srcb, dstv, etb, rows, wtabv, zbuf, macc, semg, sems):
    cid = lax.axis_index("c")
    sid = lax.axis_index("s")
    wid = cid * NS + sid

    pltpu.sync_copy(dst_hbm.at[wid], dstv)
    pltpu.sync_copy(wtab_hbm, wtabv)

    for r in range(8):
        for k in range(D // 16):
            zbuf[r, pl.ds(16 * k, 16)] = jnp.zeros((16,), jnp.float32)

    base = sid * ZB

    def _zero(i, carry):
        pltpu.sync_copy(zbuf, macc.at[pl.ds(base + i * 8, 8)])
        return carry
    lax.fori_loop(0, ZB // 8, _zero, 0)

    @pl.when(sid == NS - 1)
    def _zero_tail():
        pltpu.sync_copy(zbuf, macc.at[pl.ds(NS * ZB, 8)])
        pltpu.sync_copy(zbuf, macc.at[pl.ds(NS * ZB + 8, 8)])
    plsc.subcore_barrier()

    def _scale(local_base, b):
        def _s(g, inner):
            etk = etb[pl.ds(local_base + b * CH + 16 * g, 16)]
            wk = plsc.load_gather(wtabv, [etk])
            for j in range(16):
                s = wk[j]
                r0 = 16 * g + j
                for k in range(D // 16):
                    rows[b, r0, pl.ds(16 * k, 16)] = (
                        rows[b, r0, pl.ds(16 * k, 16)] * s)
            return inner
        lax.fori_loop(0, CH // 16, _s, 0)

    def _gather(local_base, b):
        return pltpu.async_copy(
            hn_hbm.at[srcb.at[pl.ds(local_base + b * CH, CH)]],
            rows.at[b], semg)

    def _group(local_base, cbase, nk):
        for b in range(nk):
            _gather(local_base, b)
        for b in range(nk):
            pltpu.make_async_copy(
                hn_hbm.at[srcb.at[pl.ds(local_base + b * CH, CH)]],
                rows.at[b], semg).wait()
            _scale(local_base, b)
            pltpu.async_copy(rows.at[b], macc.at[dstv.at[cbase + b]], sems,
                             add=True)
        for b in range(nk):
            pltpu.make_async_copy(rows.at[b], macc.at[dstv.at[cbase]],
                                  sems).wait()

    def _block(b7, carry):
        estart = wid * EPW + b7 * (SBLK * CH)
        pltpu.sync_copy(src_hbm.at[pl.ds(estart, SBLK * CH)], srcb)
        pltpu.sync_copy(et_hbm.at[pl.ds(estart, SBLK * CH)], etb)

        def _g5(g5, inner):
            _group(g5 * (K * CH), b7 * SBLK + g5 * K, K)
            return inner
        lax.fori_loop(0, SBLK // K, _g5, 0)
        return carry
    lax.fori_loop(0, NBLK, _block, 0)

    tstart = wid * EPW + NBLK * (SBLK * CH)
    pltpu.sync_copy(src_hbm.at[pl.ds(tstart, 5 * CH)], srcb.at[pl.ds(0, 5 * CH)])
    pltpu.sync_copy(et_hbm.at[pl.ds(tstart, 5 * CH)], etb.at[pl.ds(0, 5 * CH)])
    _group(0, NBLK * SBLK, K)
    _group(K * CH, NBLK * SBLK + K, 2)

    plsc.subcore_barrier()
    pltpu.sync_copy(macc.at[pl.ds(base, ZB)],
                    out_hbm.at[pl.ds(cid * N + base, ZB)])

    @pl.when(sid == NS - 1)
    def _copy_tail():
        pltpu.sync_copy(macc.at[pl.ds(NS * ZB, 16)],
                        out_hbm.at[pl.ds(cid * N + NS * ZB, 16)])


BR = 2000


def _pre_body(x_ref, w1t, b1, w2t, b2, hn_ref):
    h = jnp.dot(x_ref[...], w1t[...], preferred_element_type=jnp.float32) + b1[...]
    h = jnp.maximum(h, 0.0)
    h = jnp.dot(h, w2t[...], preferred_element_type=jnp.float32) + b2[...]
    mu = jnp.mean(h, axis=-1, keepdims=True)
    var = jnp.mean((h - mu) ** 2, axis=-1, keepdims=True)
    hn_ref[...] = (h - mu) * lax.rsqrt(var + 1e-5)


_pre = pl.pallas_call(
    _pre_body,
    grid=(N // BR,),
    in_specs=[
        pl.BlockSpec((BR, D), lambda i: (i, 0)),
        pl.BlockSpec((D, H), lambda i: (0, 0)),
        pl.BlockSpec((1, H), lambda i: (0, 0)),
        pl.BlockSpec((H, H), lambda i: (0, 0)),
        pl.BlockSpec((1, H), lambda i: (0, 0)),
    ],
    out_specs=pl.BlockSpec((BR, H), lambda i: (i, 0)),
    out_shape=jax.ShapeDtypeStruct((N, H), jnp.float32),
)


def _gru_body(m0, m1, d0, d1, hn_ref, wih, whh, bih, bhh, h_out, hn_out):
    deg = jnp.maximum(d0[...] + d1[...], 1.0)
    m = (m0[...] + m1[...]) / deg
    hn = hn_ref[...]
    gi = jnp.dot(m, wih[...], preferred_element_type=jnp.float32) + bih[...]
    gh = jnp.dot(hn, whh[...], preferred_element_type=jnp.float32) + bhh[...]
    r = jax.nn.sigmoid(gi[:, :H] + gh[:, :H])
    z = jax.nn.sigmoid(gi[:, H:2 * H] + gh[:, H:2 * H])
    n = jnp.tanh(gi[:, 2 * H:] + r * gh[:, 2 * H:])
    h = (1.0 - z) * n + z * hn
    h_out[...] = h
    mu = jnp.mean(h, axis=-1, keepdims=True)
    var = jnp.mean((h - mu) ** 2, axis=-1, keepdims=True)
    hn_out[...] = (h - mu) * lax.rsqrt(var + 1e-5)


_gru = pl.pallas_call(
    _gru_body,
    grid=(N // BR,),
    in_specs=[
        pl.BlockSpec((BR, D), lambda i: (i, 0)),
        pl.BlockSpec((BR, D), lambda i: (i, 0)),
        pl.BlockSpec((BR, 1), lambda i: (i, 0)),
        pl.BlockSpec((BR, 1), lambda i: (i, 0)),
        pl.BlockSpec((BR, D), lambda i: (i, 0)),
        pl.BlockSpec((H, 3 * H), lambda i: (0, 0)),
        pl.BlockSpec((H, 3 * H), lambda i: (0, 0)),
        pl.BlockSpec((1, 3 * H), lambda i: (0, 0)),
        pl.BlockSpec((1, 3 * H), lambda i: (0, 0)),
    ],
    out_specs=[
        pl.BlockSpec((BR, H), lambda i: (i, 0)),
        pl.BlockSpec((BR, H), lambda i: (i, 0)),
    ],
    out_shape=[
        jax.ShapeDtypeStruct((N, H), jnp.float32),
        jax.ShapeDtypeStruct((N, H), jnp.float32),
    ],
)


def _readout_body(h, ind, bt, wp, f1t, b1, f2t, b2, f3t, b3, out, accm, acca):
    i = pl.program_id(0)

    @pl.when(i == 0)
    def _init():
        accm[...] = jnp.zeros_like(accm)
        acca[...] = jnp.zeros_like(acca)

    alpha = 1.0 + wp[...] * ind[...]
    bio = lax.broadcasted_iota(jnp.int32, (BR, G), 1)
    oh = (bio == bt[...]).astype(jnp.float32)
    accm[...] += lax.dot_general(oh, alpha * h[...],
                                 (((0,), (0,)), ((), ())),
                                 preferred_element_type=jnp.float32)
    acca[...] += lax.dot_general(oh, alpha,
                                 (((0,), (0,)), ((), ())),
                                 preferred_element_type=jnp.float32)

    @pl.when(i == pl.num_programs(0) - 1)
    def _fin():
        xg = accm[...] / acca[...]
        o = jnp.dot(xg, f1t[...], preferred_element_type=jnp.float32) + b1[...]
        o = jnp.maximum(o, 0.0)
        o = jnp.dot(o, f2t[...], preferred_element_type=jnp.float32) + b2[...]
        o = jnp.maximum(o, 0.0)
        out[...] = jnp.dot(o, f3t[...], preferred_element_type=jnp.float32) + b3[...]


_readout = pl.pallas_call(
    _readout_body,
    grid=(N // BR,),
    in_specs=[
        pl.BlockSpec((BR, D), lambda i: (i, 0)),
        pl.BlockSpec((BR, 1), lambda i: (i, 0)),
        pl.BlockSpec((BR, 1), lambda i: (i, 0)),
        pl.BlockSpec((1, 1), lambda i: (0, 0)),
        pl.BlockSpec((H, H // 2), lambda i: (0, 0)),
        pl.BlockSpec((1, H // 2), lambda i: (0, 0)),
        pl.BlockSpec((H // 2, H // 4), lambda i: (0, 0)),
        pl.BlockSpec((1, H // 4), lambda i: (0, 0)),
        pl.BlockSpec((H // 4, 2), lambda i: (0, 0)),
        pl.BlockSpec((1, 2), lambda i: (0, 0)),
    ],
    out_specs=pl.BlockSpec((G, 2), lambda i: (0, 0)),
    out_shape=jax.ShapeDtypeStruct((G, 2), jnp.float32),
    scratch_shapes=[
        pltpu.VMEM((G, D), jnp.float32),
        pltpu.VMEM((G, 1), jnp.float32),
    ],
)


def kernel(x, edge_index, edge_type, diff_idx, batch,
           lin1_W, lin1_b, lin2_W, lin2_b, edge_type_weight,
           gru_Wih, gru_Whh, gru_bih, gru_bhh,
           fc1_W, fc1_b, fc2_W, fc2_b, fc3_W, fc3_b, w_imp):
    src1 = edge_index[0]
    dst3 = edge_index[1].reshape(NW, NCHUNK, CH)
    et1 = edge_type
    wtab = jnp.logaddexp(0.0, edge_type_weight).astype(jnp.float32)

    _prep_sc, _msg_sc = _build_sc_kernels()
    degp, ind = _prep_sc(edge_index[1], diff_idx)
    d0 = degp[:N].reshape(N, 1)
    d1 = degp[NP:NP + N].reshape(N, 1)
    indc = ind[:N].reshape(N, 1)

    hn = _pre(x, lin1_W.T, lin1_b.reshape(1, H), lin2_W.T, lin2_b.reshape(1, H))

    wih_t = gru_Wih.T
    whh_t = gru_Whh.T
    bih = gru_bih.reshape(1, 3 * H)
    bhh = gru_bhh.reshape(1, 3 * H)

    h = hn
    for _ in range(3):
        mflat = _msg_sc(hn, src1, dst3, et1, wtab)
        h, hn = _gru(mflat[:N], mflat[N:], d0, d1, hn, wih_t, whh_t, bih, bhh)

    wp = jnp.logaddexp(0.0, w_imp).reshape(1, 1)
    return _readout(h, indc, batch.reshape(N, 1), wp,
                    fc1_W.T, fc1_b.reshape(1, H // 2),
                    fc2_W.T, fc2_b.reshape(1, H // 4),
                    fc3_W.T, fc3_b.reshape(1, 2))


# speedup vs baseline: 2.6152x; 1.0189x over previous
import functools

import jax
import jax.numpy as jnp
from jax import lax
from jax.experimental import pallas as pl
from jax.experimental.pallas import tpu as pltpu
from jax.experimental.pallas import tpu_sc as plsc

N = 10000
E = 320000
D = 128
H = 128
NT = 16
G = 64
NDIFF = 1000

NC = 2
NS = 16
NW = NC * NS
EPW = E // NW
CH = 80
NCHUNK = EPW // CH
ZB = 624
K = 3
SBLK = 15
NBLK = 8
NP = 10240
CPS = NP // NS

@functools.lru_cache(maxsize=None)
def _build_sc_kernels():
    mesh = plsc.VectorSubcoreMesh(core_axis_name="c", subcore_axis_name="s")
    prep = functools.partial(
        pl.kernel,
        out_type=(
            jax.ShapeDtypeStruct((NC * NP,), jnp.float32),
            jax.ShapeDtypeStruct((NP,), jnp.float32),
        ),
        mesh=mesh,
        compiler_params=pltpu.CompilerParams(needs_layout_passes=False),
        scratch_types=[
            pltpu.VMEM((EPW,), jnp.int32),
            pltpu.VMEM((NP,), jnp.float32),
            pltpu.VMEM((CPS,), jnp.float32),
            pltpu.VMEM((CPS,), jnp.float32),
            pltpu.VMEM((NDIFF,), jnp.int32),
            pltpu.VMEM((NP,), jnp.float32),
            pltpu.VMEM_SHARED((NS * NP,), jnp.float32),
        ],
    )(_prep_sc_body)
    msg = functools.partial(
        pl.kernel,
        out_type=jax.ShapeDtypeStruct((NC * N, D), jnp.float32),
        mesh=mesh,
        compiler_params=pltpu.CompilerParams(needs_layout_passes=False),
        scratch_types=[
            pltpu.VMEM((SBLK * CH,), jnp.int32),
            pltpu.VMEM((NCHUNK, CH), jnp.int32),
            pltpu.VMEM((SBLK * CH,), jnp.int32),
            pltpu.VMEM((K, CH, D), jnp.float32),
            pltpu.VMEM((NT,), jnp.float32),
            pltpu.VMEM((8, D), jnp.float32),
            pltpu.VMEM_SHARED((N, D), jnp.float32),
            pltpu.SemaphoreType.DMA,
            pltpu.SemaphoreType.DMA,
        ],
    )(_msg_sc_body)
    return prep, msg


def _prep_sc_body(dst_hbm, diff_hbm, deg_hbm, ind_hbm,
                  dstl, degl, tmp, acc, diffv, indv, sdeg):
    cid = lax.axis_index("c")
    sid = lax.axis_index("s")
    wid = cid * NS + sid
    ones = jnp.ones((16,), jnp.float32)
    zeros = jnp.zeros((16,), jnp.float32)

    def _z(i, carry):
        degl[pl.ds(16 * i, 16)] = zeros
        return carry
    lax.fori_loop(0, NP // 16, _z, 0)

    pltpu.sync_copy(dst_hbm.at[pl.ds(wid * EPW, EPW)], dstl)

    def _hist(j, carry):
        iv = dstl[pl.ds(16 * j, 16)]
        plsc.addupdate_scatter(degl, [iv], ones)
        return carry
    lax.fori_loop(0, EPW // 16, _hist, 0)

    pltpu.sync_copy(degl, sdeg.at[pl.ds(sid * NP, NP)])
    plsc.subcore_barrier()

    col = sid * CPS

    def _za(i, carry):
        acc[pl.ds(16 * i, 16)] = zeros
        return carry
    lax.fori_loop(0, CPS // 16, _za, 0)

    for j in range(NS):
        pltpu.sync_copy(sdeg.at[pl.ds(j * NP + col, CPS)], tmp)

        def _add(i, carry):
            acc[pl.ds(16 * i, 16)] = acc[pl.ds(16 * i, 16)] + tmp[pl.ds(16 * i, 16)]
            return carry
        lax.fori_loop(0, CPS // 16, _add, 0)

    pltpu.sync_copy(acc, deg_hbm.at[pl.ds(cid * NP + col, CPS)])

    @pl.when(jnp.logical_and(cid == 0, sid == 0))
    def _ind():
        def _zi(i, carry):
            indv[pl.ds(16 * i, 16)] = zeros
            return carry
        lax.fori_loop(0, NP // 16, _zi, 0)
        pltpu.sync_copy(diff_hbm, diffv)

        def _sc(j, carry):
            iv = diffv[pl.ds(16 * j, 16)]
            plsc.store_scatter(indv, [iv], ones)
            return carry
        lax.fori_loop(0, NDIFF // 16, _sc, 0)
        iv = diffv[pl.ds(NDIFF - 16, 16)]
        plsc.store_scatter(indv, [iv], ones)
        pltpu.sync_copy(indv, ind_hbm)


def _msg_sc_body(hn_hbm, src_hbm, dst_hbm, et_hbm, wtab_hbm, out_hbm,
                 srcb, dstv, etb, rows, wtabv, zbuf, macc, semg, sems):
    cid = lax.axis_index("c")
    sid = lax.axis_index("s")
    wid = cid * NS + sid

    pltpu.sync_copy(dst_hbm.at[wid], dstv)
    pltpu.sync_copy(wtab_hbm, wtabv)

    for r in range(8):
        for k in range(D // 16):
            zbuf[r, pl.ds(16 * k, 16)] = jnp.zeros((16,), jnp.float32)

    base = sid * ZB

    def _zero(i, carry):
        pltpu.sync_copy(zbuf, macc.at[pl.ds(base + i * 8, 8)])
        return carry
    lax.fori_loop(0, ZB // 8, _zero, 0)

    @pl.when(sid == NS - 1)
    def _zero_tail():
        pltpu.sync_copy(zbuf, macc.at[pl.ds(NS * ZB, 8)])
        pltpu.sync_copy(zbuf, macc.at[pl.ds(NS * ZB + 8, 8)])
    plsc.subcore_barrier()

    def _scale(local_base, b):
        def _s(g, inner):
            etk = etb[pl.ds(local_base + b * CH + 16 * g, 16)]
            wk = plsc.load_gather(wtabv, [etk])
            for j in range(16):
                s = wk[j]
                r0 = 16 * g + j
                for k in range(D // 16):
                    rows[b, r0, pl.ds(16 * k, 16)] = (
                        rows[b, r0, pl.ds(16 * k, 16)] * s)
            return inner
        lax.fori_loop(0, CH // 16, _s, 0)

    def _gather(local_base, b):
        return pltpu.async_copy(
            hn_hbm.at[srcb.at[pl.ds(local_base + b * CH, CH)]],
            rows.at[b], semg)

    def _group(local_base, cbase, nk):
        for b in range(nk):
            _gather(local_base, b)
        for b in range(nk):
            pltpu.make_async_copy(
                hn_hbm.at[srcb.at[pl.ds(local_base + b * CH, CH)]],
                rows.at[b], semg).wait()
            _scale(local_base, b)
            pltpu.async_copy(rows.at[b], macc.at[dstv.at[cbase + b]], sems,
                             add=True)
        for b in range(nk):
            pltpu.make_async_copy(rows.at[b], macc.at[dstv.at[cbase]],
                                  sems).wait()

    def _block(b7, carry):
        estart = wid * EPW + b7 * (SBLK * CH)
        pltpu.sync_copy(src_hbm.at[pl.ds(estart, SBLK * CH)], srcb)
        pltpu.sync_copy(et_hbm.at[pl.ds(estart, SBLK * CH)], etb)

        def _g5(g5, inner):
            _group(g5 * (K * CH), b7 * SBLK + g5 * K, K)
            return inner
        lax.fori_loop(0, SBLK // K, _g5, 0)
        return carry
    lax.fori_loop(0, NBLK, _block, 0)

    tstart = wid * EPW + NBLK * (SBLK * CH)
    pltpu.sync_copy(src_hbm.at[pl.ds(tstart, 5 * CH)], srcb.at[pl.ds(0, 5 * CH)])
    pltpu.sync_copy(et_hbm.at[pl.ds(tstart, 5 * CH)], etb.at[pl.ds(0, 5 * CH)])
    _group(0, NBLK * SBLK, K)
    _group(K * CH, NBLK * SBLK + K, 2)

    plsc.subcore_barrier()
    pltpu.sync_copy(macc.at[pl.ds(base, ZB)],
                    out_hbm.at[pl.ds(cid * N + base, ZB)])

    @pl.when(sid == NS - 1)
    def _copy_tail():
        pltpu.sync_copy(macc.at[pl.ds(NS * ZB, 16)],
                        out_hbm.at[pl.ds(cid * N + NS * ZB, 16)])


BR = 2000


def _pre_body(x_ref, w1t, b1, w2t, b2, hn_ref):
    h = jnp.dot(x_ref[...], w1t[...], preferred_element_type=jnp.float32) + b1[...]
    h = jnp.maximum(h, 0.0)
    h = jnp.dot(h, w2t[...], preferred_element_type=jnp.float32) + b2[...]
    mu = jnp.mean(h, axis=-1, keepdims=True)
    var = jnp.mean((h - mu) ** 2, axis=-1, keepdims=True)
    hn_ref[...] = (h - mu) * lax.rsqrt(var + 1e-5)


_pre = pl.pallas_call(
    _pre_body,
    grid=(N // BR,),
    in_specs=[
        pl.BlockSpec((BR, D), lambda i: (i, 0)),
        pl.BlockSpec((D, H), lambda i: (0, 0)),
        pl.BlockSpec((1, H), lambda i: (0, 0)),
        pl.BlockSpec((H, H), lambda i: (0, 0)),
        pl.BlockSpec((1, H), lambda i: (0, 0)),
    ],
    out_specs=pl.BlockSpec((BR, H), lambda i: (i, 0)),
    out_shape=jax.ShapeDtypeStruct((N, H), jnp.float32),
)


def _gru_body(m0, m1, d0, d1, hn_ref, wih, whh, bih, bhh, hn_out):
    deg = jnp.maximum(d0[...] + d1[...], 1.0)
    m = (m0[...] + m1[...]) / deg
    hn = hn_ref[...]
    gi = jnp.dot(m, wih[...], preferred_element_type=jnp.float32) + bih[...]
    gh = jnp.dot(hn, whh[...], preferred_element_type=jnp.float32) + bhh[...]
    r = jax.nn.sigmoid(gi[:, :H] + gh[:, :H])
    z = jax.nn.sigmoid(gi[:, H:2 * H] + gh[:, H:2 * H])
    n = jnp.tanh(gi[:, 2 * H:] + r * gh[:, 2 * H:])
    h = (1.0 - z) * n + z * hn
    mu = jnp.mean(h, axis=-1, keepdims=True)
    var = jnp.mean((h - mu) ** 2, axis=-1, keepdims=True)
    hn_out[...] = (h - mu) * lax.rsqrt(var + 1e-5)


_gru = pl.pallas_call(
    _gru_body,
    grid=(N // BR,),
    in_specs=[
        pl.BlockSpec((BR, D), lambda i: (i, 0)),
        pl.BlockSpec((BR, D), lambda i: (i, 0)),
        pl.BlockSpec((BR, 1), lambda i: (i, 0)),
        pl.BlockSpec((BR, 1), lambda i: (i, 0)),
        pl.BlockSpec((BR, D), lambda i: (i, 0)),
        pl.BlockSpec((H, 3 * H), lambda i: (0, 0)),
        pl.BlockSpec((H, 3 * H), lambda i: (0, 0)),
        pl.BlockSpec((1, 3 * H), lambda i: (0, 0)),
        pl.BlockSpec((1, 3 * H), lambda i: (0, 0)),
    ],
    out_specs=pl.BlockSpec((BR, H), lambda i: (i, 0)),
    out_shape=jax.ShapeDtypeStruct((N, H), jnp.float32),
)


def _gru_ro_body(m0, m1, d0, d1, hn_ref, wih, whh, bih, bhh,
                 ind, bt, wp, f1t, b1, f2t, b2, f3t, b3, out, accm, acca):
    i = pl.program_id(0)

    @pl.when(i == 0)
    def _init():
        accm[...] = jnp.zeros_like(accm)
        acca[...] = jnp.zeros_like(acca)

    deg = jnp.maximum(d0[...] + d1[...], 1.0)
    m = (m0[...] + m1[...]) / deg
    hn = hn_ref[...]
    gi = jnp.dot(m, wih[...], preferred_element_type=jnp.float32) + bih[...]
    gh = jnp.dot(hn, whh[...], preferred_element_type=jnp.float32) + bhh[...]
    r = jax.nn.sigmoid(gi[:, :H] + gh[:, :H])
    z = jax.nn.sigmoid(gi[:, H:2 * H] + gh[:, H:2 * H])
    n = jnp.tanh(gi[:, 2 * H:] + r * gh[:, 2 * H:])
    h = (1.0 - z) * n + z * hn

    alpha = 1.0 + wp[...] * ind[...]
    bio = lax.broadcasted_iota(jnp.int32, (BR, G), 1)
    oh = (bio == bt[...]).astype(jnp.float32)
    accm[...] += lax.dot_general(oh, alpha * h,
                                 (((0,), (0,)), ((), ())),
                                 preferred_element_type=jnp.float32)
    acca[...] += lax.dot_general(oh, alpha,
                                 (((0,), (0,)), ((), ())),
                                 preferred_element_type=jnp.float32)

    @pl.when(i == pl.num_programs(0) - 1)
    def _fin():
        xg = accm[...] / acca[...]
        o = jnp.dot(xg, f1t[...], preferred_element_type=jnp.float32) + b1[...]
        o = jnp.maximum(o, 0.0)
        o = jnp.dot(o, f2t[...], preferred_element_type=jnp.float32) + b2[...]
        o = jnp.maximum(o, 0.0)
        out[...] = jnp.dot(o, f3t[...], preferred_element_type=jnp.float32) + b3[...]


_gru_ro = pl.pallas_call(
    _gru_ro_body,
    grid=(N // BR,),
    in_specs=[
        pl.BlockSpec((BR, D), lambda i: (i, 0)),
        pl.BlockSpec((BR, D), lambda i: (i, 0)),
        pl.BlockSpec((BR, 1), lambda i: (i, 0)),
        pl.BlockSpec((BR, 1), lambda i: (i, 0)),
        pl.BlockSpec((BR, D), lambda i: (i, 0)),
        pl.BlockSpec((H, 3 * H), lambda i: (0, 0)),
        pl.BlockSpec((H, 3 * H), lambda i: (0, 0)),
        pl.BlockSpec((1, 3 * H), lambda i: (0, 0)),
        pl.BlockSpec((1, 3 * H), lambda i: (0, 0)),
        pl.BlockSpec((BR, 1), lambda i: (i, 0)),
        pl.BlockSpec((BR, 1), lambda i: (i, 0)),
        pl.BlockSpec((1, 1), lambda i: (0, 0)),
        pl.BlockSpec((H, H // 2), lambda i: (0, 0)),
        pl.BlockSpec((1, H // 2), lambda i: (0, 0)),
        pl.BlockSpec((H // 2, H // 4), lambda i: (0, 0)),
        pl.BlockSpec((1, H // 4), lambda i: (0, 0)),
        pl.BlockSpec((H // 4, 2), lambda i: (0, 0)),
        pl.BlockSpec((1, 2), lambda i: (0, 0)),
    ],
    out_specs=pl.BlockSpec((G, 2), lambda i: (0, 0)),
    out_shape=jax.ShapeDtypeStruct((G, 2), jnp.float32),
    scratch_shapes=[
        pltpu.VMEM((G, D), jnp.float32),
        pltpu.VMEM((G, 1), jnp.float32),
    ],
)


def kernel(x, edge_index, edge_type, diff_idx, batch,
           lin1_W, lin1_b, lin2_W, lin2_b, edge_type_weight,
           gru_Wih, gru_Whh, gru_bih, gru_bhh,
           fc1_W, fc1_b, fc2_W, fc2_b, fc3_W, fc3_b, w_imp):
    src1 = edge_index[0]
    dst3 = edge_index[1].reshape(NW, NCHUNK, CH)
    et1 = edge_type
    wtab = jnp.logaddexp(0.0, edge_type_weight).astype(jnp.float32)

    _prep_sc, _msg_sc = _build_sc_kernels()
    degp, ind = _prep_sc(edge_index[1], diff_idx)
    d0 = degp[:N].reshape(N, 1)
    d1 = degp[NP:NP + N].reshape(N, 1)
    indc = ind[:N].reshape(N, 1)

    hn = _pre(x, lin1_W.T, lin1_b.reshape(1, H), lin2_W.T, lin2_b.reshape(1, H))

    wih_t = gru_Wih.T
    whh_t = gru_Whh.T
    bih = gru_bih.reshape(1, 3 * H)
    bhh = gru_bhh.reshape(1, 3 * H)

    for _ in range(2):
        mflat = _msg_sc(hn, src1, dst3, et1, wtab)
        hn = _gru(mflat[:N], mflat[N:], d0, d1, hn, wih_t, whh_t, bih, bhh)

    wp = jnp.logaddexp(0.0, w_imp).reshape(1, 1)
    mflat = _msg_sc(hn, src1, dst3, et1, wtab)
    return _gru_ro(mflat[:N], mflat[N:], d0, d1, hn, wih_t, whh_t, bih, bhh,
                   indc, batch.reshape(N, 1), wp,
                   fc1_W.T, fc1_b.reshape(1, H // 2),
                   fc2_W.T, fc2_b.reshape(1, H // 4),
                   fc3_W.T, fc3_b.reshape(1, 2))
